# Initial kernel scaffold; baseline (speedup 1.0000x reference)
import jax
import jax.numpy as jnp
from jax.experimental import pallas as pl


def kernel(x, edge_index, W1, b1, W2, b2):
    raise NotImplementedError("write your pallas kernel here")


# speedup vs baseline: 17.8055x; 17.8055x over previous
import functools

import jax
import jax.numpy as jnp
from jax import lax
from jax.experimental import pallas as pl
from jax.experimental.pallas import tpu as pltpu
from jax.experimental.pallas import tpu_sc as plsc

N = 10000
D = 128
P = 10112
E = 320000
C = 128
NC = 2
NS = 16
NW = NC * NS
CH = 79
E_PAD = NW * C * CH
RPT = P // NS
RSTG = RPT

_MESH = plsc.VectorSubcoreMesh(core_axis_name="c", subcore_axis_name="s")
_f32 = jnp.float32


@functools.partial(
    pl.kernel,
    out_type=jax.ShapeDtypeStruct((2 * P,), _f32),
    mesh=_MESH,
    scratch_types=[
        pltpu.VMEM((C,), jnp.int32),
        pltpu.VMEM((C,), _f32),
        pltpu.VMEM((RPT,), _f32),
        pltpu.VMEM_SHARED((P,), _f32),
    ],
)
def _deg_kernel(dst_hbm, ones_hbm, zeros_hbm, out_hbm,
                didx_v, ones_v, stage_v, deg_sh):
    c = lax.axis_index("c")
    s = lax.axis_index("s")
    wid = s * NC + c
    lo = s * RPT

    @pl.when(c == 0)
    def _():
        pltpu.sync_copy(ones_hbm.at[pl.ds(lo, RPT)], stage_v)

    @pl.when(c == 1)
    def _():
        pltpu.sync_copy(zeros_hbm.at[pl.ds(lo, RPT)], stage_v)

    pltpu.sync_copy(stage_v, deg_sh.at[pl.ds(lo, RPT)])

    for j in range(C // 16):
        ones_v[pl.ds(j * 16, 16)] = jnp.full((16,), 1.0, _f32)

    plsc.subcore_barrier()

    def body(i, carry):
        base = (wid * CH + i) * C
        pltpu.sync_copy(dst_hbm.at[pl.ds(base, C)], didx_v)
        pltpu.sync_copy(ones_v, deg_sh.at[didx_v], add=True)
        return carry

    lax.fori_loop(0, CH, body, 0)
    plsc.subcore_barrier()

    pltpu.sync_copy(deg_sh.at[pl.ds(lo, RPT)], stage_v)
    pltpu.sync_copy(stage_v, out_hbm.at[pl.ds(c * P + lo, RPT)])


DH = D // 2


@functools.partial(
    pl.kernel,
    out_type=(jax.ShapeDtypeStruct((2 * P, DH), _f32),
              jax.ShapeDtypeStruct((2 * P, DH), _f32)),
    mesh=_MESH,
    scratch_types=[
        pltpu.VMEM((C,), jnp.int32),
        pltpu.VMEM((C,), jnp.int32),
        pltpu.VMEM((C, DH), _f32),
        pltpu.VMEM((RSTG, DH), _f32),
        pltpu.VMEM_SHARED((P, DH), _f32),
        pltpu.SemaphoreType.DMA,
    ],
    compiler_params=pltpu.CompilerParams(use_tc_tiling_on_sc=False),
)
def _agg1_kernel(src_hbm, dst_hbm, ulo_hbm, uhi_hbm, zeros_hbm,
                 outlo_hbm, outhi_hbm,
                 sidx_v, didx_v, rows_v, stage_v, acc_sh, sem):
    c = lax.axis_index("c")
    s = lax.axis_index("s")
    wid = s * NC + c
    lo = s * RPT

    for u_hbm, out_hbm in ((ulo_hbm, outlo_hbm), (uhi_hbm, outhi_hbm)):
        @pl.when(c == 0)
        def _():
            pltpu.sync_copy(u_hbm.at[pl.ds(lo, RPT)], stage_v)

        @pl.when(c == 1)
        def _():
            pltpu.sync_copy(zeros_hbm.at[pl.ds(lo, RPT)], stage_v)

        pltpu.sync_copy(stage_v, acc_sh.at[pl.ds(lo, RPT)])

        plsc.subcore_barrier()

        def body(i, carry):
            base = (wid * CH + i) * C
            pltpu.sync_copy(src_hbm.at[pl.ds(base, C)], sidx_v)
            pltpu.sync_copy(dst_hbm.at[pl.ds(base, C)], didx_v)
            pltpu.async_copy(u_hbm.at[sidx_v], rows_v, sem).wait()
            pltpu.sync_copy(rows_v, acc_sh.at[didx_v], add=True)
            return carry

        lax.fori_loop(0, CH, body, 0)
        plsc.subcore_barrier()

        pltpu.sync_copy(acc_sh.at[pl.ds(lo, RPT)], stage_v)
        pltpu.sync_copy(stage_v, out_hbm.at[pl.ds(c * P + lo, RPT)])


@functools.partial(
    pl.kernel,
    out_type=jax.ShapeDtypeStruct((2 * P,), _f32),
    mesh=_MESH,
    scratch_types=[
        pltpu.VMEM((C,), jnp.int32),
        pltpu.VMEM((C,), jnp.int32),
        pltpu.VMEM((C,), _f32),
        pltpu.VMEM((P,), _f32),
        pltpu.VMEM((RPT,), _f32),
        pltpu.VMEM_SHARED((P,), _f32),
    ],
    compiler_params=pltpu.CompilerParams(needs_layout_passes=False),
)
def _agg2_kernel(src_hbm, dst_hbm, u2_hbm, zeros_hbm, out_hbm,
                 sidx_v, didx_v, vals_v, u2_v, stage_v, acc_sh):
    c = lax.axis_index("c")
    s = lax.axis_index("s")
    wid = s * NC + c
    lo = s * RPT

    pltpu.sync_copy(u2_hbm, u2_v)

    @pl.when(c == 0)
    def _():
        pltpu.sync_copy(u2_v.at[pl.ds(lo, RPT)], acc_sh.at[pl.ds(lo, RPT)])

    @pl.when(c == 1)
    def _():
        pltpu.sync_copy(zeros_hbm.at[pl.ds(lo, RPT)], stage_v)
        pltpu.sync_copy(stage_v, acc_sh.at[pl.ds(lo, RPT)])

    plsc.subcore_barrier()

    def body(i, carry):
        base = (wid * CH + i) * C
        pltpu.sync_copy(src_hbm.at[pl.ds(base, C)], sidx_v)
        pltpu.sync_copy(dst_hbm.at[pl.ds(base, C)], didx_v)
        for j in range(C // 16):
            sv = sidx_v[pl.ds(j * 16, 16)]
            vals_v[pl.ds(j * 16, 16)] = plsc.load_gather(u2_v, [sv])
        pltpu.sync_copy(vals_v, acc_sh.at[didx_v], add=True)
        return carry

    lax.fori_loop(0, CH, body, 0)
    plsc.subcore_barrier()

    pltpu.sync_copy(acc_sh.at[pl.ds(lo, RPT)], stage_v)
    pltpu.sync_copy(stage_v, out_hbm.at[pl.ds(c * P + lo, RPT)])


def _mm_scale_body(x_ref, w_ref, deg_ref, ulo_ref, uhi_ref, dinv_ref):
    deg = deg_ref[0, :] + deg_ref[1, :]
    dinv = lax.rsqrt(deg)
    u = dinv[:, None] * jnp.dot(x_ref[...], w_ref[...],
                                preferred_element_type=_f32)
    ulo_ref[...] = u[:, :DH]
    uhi_ref[...] = u[:, DH:]
    dinv_ref[...] = dinv


def _relu_mv_body(plo_ref, phi_ref, dinv_ref, b1_ref, w2_ref, u2_ref):
    ssum = jnp.concatenate(
        [plo_ref[0] + plo_ref[1], phi_ref[0] + phi_ref[1]], axis=1)
    dinv = dinv_ref[...]
    h = jnp.maximum(dinv[:, None] * ssum + b1_ref[...][None, :], 0.0)
    z = jnp.dot(h, w2_ref[...], preferred_element_type=_f32)
    u2_ref[...] = dinv * z[:, 0]


def _final_body(q_ref, dinv_ref, b2_ref, out_ref):
    v = dinv_ref[...] * (q_ref[0] + q_ref[1]) + b2_ref[0]
    out_ref[...] = jax.nn.sigmoid(v)


def kernel(x, edge_index, W1, b1, W2, b2):
    src = edge_index[0].astype(jnp.int32)
    dst = edge_index[1].astype(jnp.int32)

    npad = E_PAD - E
    pad_idx = (N + (jnp.arange(npad, dtype=jnp.int32) % (P - N)))
    src_p = jnp.concatenate([src, pad_idx])
    dst_p = jnp.concatenate([dst, pad_idx])

    x_p = jnp.pad(x, ((0, P - N), (0, 0)))
    ones1 = jnp.ones((P,), _f32)
    zeros1 = jnp.zeros((P,), _f32)
    zeros2 = jnp.zeros((P, DH), _f32)

    deg_p = _deg_kernel(dst_p, ones1, zeros1).reshape(2, P)

    u_lo, u_hi, dinv = pl.pallas_call(
        _mm_scale_body,
        out_shape=(jax.ShapeDtypeStruct((P, DH), _f32),
                   jax.ShapeDtypeStruct((P, DH), _f32),
                   jax.ShapeDtypeStruct((P,), _f32)),
    )(x_p, W1, deg_p)

    p_lo, p_hi = _agg1_kernel(src_p, dst_p, u_lo, u_hi, zeros2)

    u2 = pl.pallas_call(
        _relu_mv_body,
        out_shape=jax.ShapeDtypeStruct((P,), _f32),
    )(p_lo.reshape(2, P, DH), p_hi.reshape(2, P, DH), dinv, b1, W2)

    part2 = _agg2_kernel(src_p, dst_p, u2, zeros1).reshape(2, P)

    out_pad = pl.pallas_call(
        _final_body,
        out_shape=jax.ShapeDtypeStruct((P,), _f32),
    )(part2, dinv, b2)

    return out_pad[:N].reshape(N, 1)


# speedup vs baseline: 33.6429x; 1.8895x over previous
import functools

import jax
import jax.numpy as jnp
from jax import lax
from jax.experimental import pallas as pl
from jax.experimental.pallas import tpu as pltpu
from jax.experimental.pallas import tpu_sc as plsc

N = 10000
D = 128
DH = D // 2
P = 10112
E = 320000
C = 128
NC = 2
NS = 16
NW = NC * NS
CH = 80
E_PAD = NW * C * CH
NCHUNK = NW * CH
RPT = P // NS

_MESH = plsc.VectorSubcoreMesh(core_axis_name="c", subcore_axis_name="s")
_f32 = jnp.float32


@functools.partial(
    pl.kernel,
    out_type=jax.ShapeDtypeStruct((2 * P,), _f32),
    mesh=_MESH,
    scratch_types=[
        pltpu.VMEM((C,), jnp.int32),
        pltpu.VMEM((C,), jnp.int32),
        pltpu.VMEM((C,), _f32),
        pltpu.VMEM((RPT,), _f32),
        pltpu.VMEM_SHARED((P,), _f32),
        pltpu.SemaphoreType.DMA,
        pltpu.SemaphoreType.DMA,
    ],
)
def _deg_kernel(idxp_hbm, ones_hbm, zeros_hbm, out_hbm,
                ib0, ib1, ones_v, stage_v, deg_sh, sem0, sem1):
    c = lax.axis_index("c")
    s = lax.axis_index("s")
    wid = s * NC + c
    lo = s * RPT
    cbase = wid * CH

    @pl.when(c == 0)
    def _():
        pltpu.sync_copy(ones_hbm.at[pl.ds(lo, RPT)], stage_v)

    @pl.when(c == 1)
    def _():
        pltpu.sync_copy(zeros_hbm.at[pl.ds(lo, RPT)], stage_v)

    pltpu.sync_copy(stage_v, deg_sh.at[pl.ds(lo, RPT)])

    for j in range(C // 16):
        ones_v[pl.ds(j * 16, 16)] = jnp.full((16,), 1.0, _f32)

    plsc.subcore_barrier()

    pltpu.async_copy(idxp_hbm.at[cbase + 0, 1], ib0, sem0)
    pltpu.async_copy(idxp_hbm.at[cbase + 1, 1], ib1, sem1)

    def body(i, carry):
        k = i * 2
        for b, (ib, sem) in enumerate(((ib0, sem0), (ib1, sem1))):
            m = k + b
            pltpu.make_async_copy(idxp_hbm.at[0, 1], ib, sem).wait()
            pltpu.sync_copy(ones_v, deg_sh.at[ib], add=True)

            @pl.when(m + 2 < CH)
            def _():
                pltpu.async_copy(idxp_hbm.at[cbase + m + 2, 1], ib, sem)
        return carry

    lax.fori_loop(0, CH // 2, body, 0)
    plsc.subcore_barrier()

    pltpu.sync_copy(deg_sh.at[pl.ds(lo, RPT)], stage_v)
    pltpu.sync_copy(stage_v, out_hbm.at[pl.ds(c * P + lo, RPT)])


@functools.partial(
    pl.kernel,
    out_type=(jax.ShapeDtypeStruct((2 * P, DH), _f32),
              jax.ShapeDtypeStruct((2 * P, DH), _f32)),
    mesh=_MESH,
    scratch_types=[
        pltpu.VMEM((2, C), jnp.int32),
        pltpu.VMEM((2, C), jnp.int32),
        pltpu.VMEM((C, DH), _f32),
        pltpu.VMEM((C, DH), _f32),
        pltpu.VMEM((RPT, DH), _f32),
        pltpu.VMEM_SHARED((P, DH), _f32),
        pltpu.SemaphoreType.DMA,
        pltpu.SemaphoreType.DMA,
    ],
    compiler_params=pltpu.CompilerParams(use_tc_tiling_on_sc=False),
)
def _agg1_kernel(idxp_hbm, ulo_hbm, uhi_hbm, zeros_hbm,
                 outlo_hbm, outhi_hbm,
                 ib0, ib1, rows0, rows1, stage_v, acc_sh, sem0, sem1):
    c = lax.axis_index("c")
    s = lax.axis_index("s")
    wid = s * NC + c
    lo = s * RPT
    cbase = wid * CH

    for u_hbm, out_hbm in ((ulo_hbm, outlo_hbm), (uhi_hbm, outhi_hbm)):
        @pl.when(c == 0)
        def _():
            pltpu.sync_copy(u_hbm.at[pl.ds(lo, RPT)], stage_v)

        @pl.when(c == 1)
        def _():
            pltpu.sync_copy(zeros_hbm.at[pl.ds(lo, RPT)], stage_v)

        pltpu.sync_copy(stage_v, acc_sh.at[pl.ds(lo, RPT)])

        plsc.subcore_barrier()

        pltpu.sync_copy(idxp_hbm.at[cbase + 0], ib0)
        pltpu.async_copy(u_hbm.at[ib0.at[0]], rows0, sem0)
        pltpu.sync_copy(idxp_hbm.at[cbase + 1], ib1)
        pltpu.async_copy(u_hbm.at[ib1.at[0]], rows1, sem1)

        def body(i, carry):
            k = i * 2
            for b, (ib, rows, sem) in enumerate(
                    ((ib0, rows0, sem0), (ib1, rows1, sem1))):
                m = k + b
                pltpu.make_async_copy(u_hbm.at[pl.ds(0, C)], rows, sem).wait()
                pltpu.sync_copy(rows, acc_sh.at[ib.at[1]], add=True)

                @pl.when(m + 2 < CH)
                def _():
                    pltpu.sync_copy(idxp_hbm.at[cbase + m + 2], ib)
                    pltpu.async_copy(u_hbm.at[ib.at[0]], rows, sem)
            return carry

        lax.fori_loop(0, CH // 2, body, 0)
        plsc.subcore_barrier()

        pltpu.sync_copy(acc_sh.at[pl.ds(lo, RPT)], stage_v)
        pltpu.sync_copy(stage_v, out_hbm.at[pl.ds(c * P + lo, RPT)])


@functools.partial(
    pl.kernel,
    out_type=jax.ShapeDtypeStruct((2 * P,), _f32),
    mesh=_MESH,
    scratch_types=[
        pltpu.VMEM((2, C), jnp.int32),
        pltpu.VMEM((2, C), jnp.int32),
        pltpu.VMEM((C,), _f32),
        pltpu.VMEM((P,), _f32),
        pltpu.VMEM((RPT,), _f32),
        pltpu.VMEM_SHARED((P,), _f32),
        pltpu.SemaphoreType.DMA,
        pltpu.SemaphoreType.DMA,
    ],
    compiler_params=pltpu.CompilerParams(needs_layout_passes=False),
)
def _agg2_kernel(idxp_hbm, u2_hbm, zeros_hbm, out_hbm,
                 ib0, ib1, vals_v, u2_v, stage_v, acc_sh, sem0, sem1):
    c = lax.axis_index("c")
    s = lax.axis_index("s")
    wid = s * NC + c
    lo = s * RPT
    cbase = wid * CH

    pltpu.sync_copy(u2_hbm, u2_v)

    @pl.when(c == 0)
    def _():
        pltpu.sync_copy(u2_v.at[pl.ds(lo, RPT)], acc_sh.at[pl.ds(lo, RPT)])

    @pl.when(c == 1)
    def _():
        pltpu.sync_copy(zeros_hbm.at[pl.ds(lo, RPT)], stage_v)
        pltpu.sync_copy(stage_v, acc_sh.at[pl.ds(lo, RPT)])

    plsc.subcore_barrier()

    pltpu.async_copy(idxp_hbm.at[cbase + 0], ib0, sem0)
    pltpu.async_copy(idxp_hbm.at[cbase + 1], ib1, sem1)

    def body(i, carry):
        k = i * 2
        for b, (ib, sem) in enumerate(((ib0, sem0), (ib1, sem1))):
            m = k + b
            pltpu.make_async_copy(idxp_hbm.at[0], ib, sem).wait()
            for j in range(C // 16):
                sv = ib[0, pl.ds(j * 16, 16)]
                vals_v[pl.ds(j * 16, 16)] = plsc.load_gather(u2_v, [sv])
            pltpu.sync_copy(vals_v, acc_sh.at[ib.at[1]], add=True)

            @pl.when(m + 2 < CH)
            def _():
                pltpu.async_copy(idxp_hbm.at[cbase + m + 2], ib, sem)
        return carry

    lax.fori_loop(0, CH // 2, body, 0)
    plsc.subcore_barrier()

    pltpu.sync_copy(acc_sh.at[pl.ds(lo, RPT)], stage_v)
    pltpu.sync_copy(stage_v, out_hbm.at[pl.ds(c * P + lo, RPT)])


def _mm_scale_body(x_ref, w_ref, deg_ref, ulo_ref, uhi_ref, dinv_ref):
    deg = deg_ref[0, :] + deg_ref[1, :]
    dinv = lax.rsqrt(deg)
    u = dinv[:, None] * jnp.dot(x_ref[...], w_ref[...],
                                preferred_element_type=_f32)
    ulo_ref[...] = u[:, :DH]
    uhi_ref[...] = u[:, DH:]
    dinv_ref[...] = dinv


def _relu_mv_body(plo_ref, phi_ref, dinv_ref, b1_ref, w2_ref, u2_ref):
    ssum = jnp.concatenate(
        [plo_ref[0] + plo_ref[1], phi_ref[0] + phi_ref[1]], axis=1)
    dinv = dinv_ref[...]
    h = jnp.maximum(dinv[:, None] * ssum + b1_ref[...][None, :], 0.0)
    z = jnp.dot(h, w2_ref[...], preferred_element_type=_f32)
    u2_ref[...] = dinv * z[:, 0]


def _final_body(q_ref, dinv_ref, b2_ref, out_ref):
    v = dinv_ref[...] * (q_ref[0] + q_ref[1]) + b2_ref[0]
    out_ref[...] = jax.nn.sigmoid(v)


def kernel(x, edge_index, W1, b1, W2, b2):
    src = edge_index[0].astype(jnp.int32)
    dst = edge_index[1].astype(jnp.int32)

    npad = E_PAD - E
    pad_idx = (N + (jnp.arange(npad, dtype=jnp.int32) % (P - N)))
    src_p = jnp.concatenate([src, pad_idx])
    dst_p = jnp.concatenate([dst, pad_idx])
    idxp = jnp.stack(
        [src_p.reshape(NCHUNK, C), dst_p.reshape(NCHUNK, C)], axis=1)

    x_p = jnp.pad(x, ((0, P - N), (0, 0)))
    ones1 = jnp.ones((P,), _f32)
    zeros1 = jnp.zeros((P,), _f32)
    zeros2 = jnp.zeros((P, DH), _f32)

    deg_p = _deg_kernel(idxp, ones1, zeros1).reshape(2, P)

    u_lo, u_hi, dinv = pl.pallas_call(
        _mm_scale_body,
        out_shape=(jax.ShapeDtypeStruct((P, DH), _f32),
                   jax.ShapeDtypeStruct((P, DH), _f32),
                   jax.ShapeDtypeStruct((P,), _f32)),
    )(x_p, W1, deg_p)

    p_lo, p_hi = _agg1_kernel(idxp, u_lo, u_hi, zeros2)

    u2 = pl.pallas_call(
        _relu_mv_body,
        out_shape=jax.ShapeDtypeStruct((P,), _f32),
    )(p_lo.reshape(2, P, DH), p_hi.reshape(2, P, DH), dinv, b1, W2)

    part2 = _agg2_kernel(idxp, u2, zeros1).reshape(2, P)

    out_pad = pl.pallas_call(
        _final_body,
        out_shape=jax.ShapeDtypeStruct((P,), _f32),
    )(part2, dinv, b2)

    return out_pad[:N].reshape(N, 1)


# speedup vs baseline: 44.2176x; 1.3143x over previous
import functools

import jax
import jax.numpy as jnp
from jax import lax
from jax.experimental import pallas as pl
from jax.experimental.pallas import tpu as pltpu
from jax.experimental.pallas import tpu_sc as plsc

N = 10000
D = 128
DH = D // 2
P = 10112
E = 320000
C = 128
NC = 2
NS = 16
NW = NC * NS
CH = 80
E_PAD = NW * C * CH
NCHUNK = NW * CH
RPT = P // NS

_MESH = plsc.VectorSubcoreMesh(core_axis_name="c", subcore_axis_name="s")
_f32 = jnp.float32


@functools.partial(
    pl.kernel,
    out_type=jax.ShapeDtypeStruct((2 * P,), _f32),
    mesh=_MESH,
    scratch_types=[
        pltpu.VMEM((CH, 2, C), jnp.int32),
        pltpu.VMEM((C,), _f32),
        pltpu.VMEM((RPT,), _f32),
        pltpu.VMEM_SHARED((P,), _f32),
        pltpu.SemaphoreType.DMA,
    ],
)
def _deg_kernel(idxp_hbm, ones_hbm, zeros_hbm, out_hbm,
                idx_v, ones_v, stage_v, deg_sh, sem):
    c = lax.axis_index("c")
    s = lax.axis_index("s")
    wid = s * NC + c
    lo = s * RPT
    cbase = wid * CH

    @pl.when(c == 0)
    def _():
        pltpu.sync_copy(ones_hbm.at[pl.ds(lo, RPT)], stage_v)

    @pl.when(c == 1)
    def _():
        pltpu.sync_copy(zeros_hbm.at[pl.ds(lo, RPT)], stage_v)

    pltpu.sync_copy(stage_v, deg_sh.at[pl.ds(lo, RPT)])

    pltpu.sync_copy(idxp_hbm.at[pl.ds(cbase, CH)], idx_v)

    for j in range(C // 16):
        ones_v[pl.ds(j * 16, 16)] = jnp.full((16,), 1.0, _f32)

    plsc.subcore_barrier()

    def body(m, carry):
        pltpu.async_copy(ones_v, deg_sh.at[idx_v.at[m, 1]], sem, add=True)
        return carry

    lax.fori_loop(0, CH, body, 0)

    def drain(m, carry):
        pltpu.make_async_copy(ones_v, deg_sh.at[idx_v.at[0, 1]], sem).wait()
        return carry

    lax.fori_loop(0, CH, drain, 0)
    plsc.subcore_barrier()

    pltpu.sync_copy(deg_sh.at[pl.ds(lo, RPT)], stage_v)
    pltpu.sync_copy(stage_v, out_hbm.at[pl.ds(c * P + lo, RPT)])


@functools.partial(
    pl.kernel,
    out_type=(jax.ShapeDtypeStruct((2 * P, DH), _f32),
              jax.ShapeDtypeStruct((2 * P, DH), _f32)),
    mesh=_MESH,
    scratch_types=[
        pltpu.VMEM((CH, 2, C), jnp.int32),
        pltpu.VMEM((C, DH), _f32),
        pltpu.VMEM((C, DH), _f32),
        pltpu.VMEM((C, DH), _f32),
        pltpu.VMEM((C, DH), _f32),
        pltpu.VMEM_SHARED((P, DH), _f32),
        pltpu.SemaphoreType.DMA,
        pltpu.SemaphoreType.DMA,
        pltpu.SemaphoreType.DMA,
        pltpu.SemaphoreType.DMA,
        pltpu.SemaphoreType.DMA,
        pltpu.SemaphoreType.DMA,
        pltpu.SemaphoreType.DMA,
        pltpu.SemaphoreType.DMA,
    ],
    compiler_params=pltpu.CompilerParams(use_tc_tiling_on_sc=False),
)
def _agg1_kernel(idxp_hbm, ulo_hbm, uhi_hbm, zeros_hbm,
                 outlo_hbm, outhi_hbm,
                 idx_v, rows0, rows1, rows2, rows3, acc_sh,
                 g0, g1, g2, g3, s0, s1, s2, s3):
    c = lax.axis_index("c")
    s = lax.axis_index("s")
    wid = s * NC + c
    lo = s * RPT
    cbase = wid * CH

    rows = (rows0, rows1, rows2, rows3)
    gsem = (g0, g1, g2, g3)
    ssem = (s0, s1, s2, s3)
    _PIECES = ((0, C), (C, C), (2 * C, C), (3 * C, C), (4 * C, RPT - 4 * C))

    pltpu.sync_copy(idxp_hbm.at[pl.ds(cbase, CH)], idx_v)

    for u_hbm, out_hbm in ((ulo_hbm, outlo_hbm), (uhi_hbm, outhi_hbm)):
        init_src = (u_hbm, zeros_hbm)
        for ci in range(2):
            @pl.when(c == ci)
            def _():
                for off, ln in _PIECES:
                    pltpu.sync_copy(init_src[ci].at[pl.ds(lo + off, ln)],
                                    rows[0].at[pl.ds(0, ln)])
                    pltpu.sync_copy(rows[0].at[pl.ds(0, ln)],
                                    acc_sh.at[pl.ds(lo + off, ln)])

        plsc.subcore_barrier()

        pltpu.async_copy(u_hbm.at[idx_v.at[0, 0]], rows0, g0)
        pltpu.async_copy(u_hbm.at[idx_v.at[1, 0]], rows1, g1)

        def body(i, carry):
            k = i * 4
            for b in range(4):
                m = k + b
                bn = (b + 2) % 4
                pltpu.make_async_copy(
                    u_hbm.at[pl.ds(0, C)], rows[b], gsem[b]).wait()
                pltpu.async_copy(
                    rows[b], acc_sh.at[idx_v.at[m, 1]], ssem[b], add=True)

                @pl.when(m + 2 < CH)
                def _():
                    @pl.when(m >= 2)
                    def _():
                        pltpu.make_async_copy(
                            rows[bn], acc_sh.at[idx_v.at[0, 1]],
                            ssem[bn]).wait()

                    pltpu.async_copy(
                        u_hbm.at[idx_v.at[m + 2, 0]], rows[bn], gsem[bn])
            return carry

        lax.fori_loop(0, CH // 4, body, 0)
        for b in range(4):
            pltpu.make_async_copy(
                rows[b], acc_sh.at[idx_v.at[0, 1]], ssem[b]).wait()
        plsc.subcore_barrier()

        for off, ln in _PIECES:
            pltpu.sync_copy(acc_sh.at[pl.ds(lo + off, ln)],
                            rows[0].at[pl.ds(0, ln)])
            pltpu.sync_copy(rows[0].at[pl.ds(0, ln)],
                            out_hbm.at[pl.ds(c * P + lo + off, ln)])
        plsc.subcore_barrier()


@functools.partial(
    pl.kernel,
    out_type=jax.ShapeDtypeStruct((2 * P,), _f32),
    mesh=_MESH,
    scratch_types=[
        pltpu.VMEM((CH, 2, C), jnp.int32),
        pltpu.VMEM((C,), _f32),
        pltpu.VMEM((C,), _f32),
        pltpu.VMEM((P,), _f32),
        pltpu.VMEM((RPT,), _f32),
        pltpu.VMEM_SHARED((P,), _f32),
        pltpu.SemaphoreType.DMA,
        pltpu.SemaphoreType.DMA,
    ],
    compiler_params=pltpu.CompilerParams(needs_layout_passes=False),
)
def _agg2_kernel(idxp_hbm, u2_hbm, zeros_hbm, out_hbm,
                 idx_v, vals0, vals1, u2_v, stage_v, acc_sh, s0, s1):
    c = lax.axis_index("c")
    s = lax.axis_index("s")
    wid = s * NC + c
    lo = s * RPT
    cbase = wid * CH

    pltpu.sync_copy(u2_hbm, u2_v)
    pltpu.sync_copy(idxp_hbm.at[pl.ds(cbase, CH)], idx_v)

    @pl.when(c == 0)
    def _():
        pltpu.sync_copy(u2_v.at[pl.ds(lo, RPT)], acc_sh.at[pl.ds(lo, RPT)])

    @pl.when(c == 1)
    def _():
        pltpu.sync_copy(zeros_hbm.at[pl.ds(lo, RPT)], stage_v)
        pltpu.sync_copy(stage_v, acc_sh.at[pl.ds(lo, RPT)])

    plsc.subcore_barrier()

    vals = (vals0, vals1)
    ssem = (s0, s1)

    def body(i, carry):
        k = i * 2
        for b in range(2):
            m = k + b
            for j in range(C // 16):
                sv = idx_v[m, 0, pl.ds(j * 16, 16)]
                vals[b][pl.ds(j * 16, 16)] = plsc.load_gather(u2_v, [sv])

            @pl.when(m >= 2)
            def _():
                pltpu.make_async_copy(
                    vals[b], acc_sh.at[idx_v.at[0, 1]], ssem[b]).wait()

            pltpu.async_copy(
                vals[b], acc_sh.at[idx_v.at[m, 1]], ssem[b], add=True)
        return carry

    lax.fori_loop(0, CH // 2, body, 0)
    for b in range(2):
        pltpu.make_async_copy(vals[b], acc_sh.at[idx_v.at[0, 1]], ssem[b]).wait()
    plsc.subcore_barrier()

    pltpu.sync_copy(acc_sh.at[pl.ds(lo, RPT)], stage_v)
    pltpu.sync_copy(stage_v, out_hbm.at[pl.ds(c * P + lo, RPT)])


def _mm_scale_body(x_ref, w_ref, deg_ref, ulo_ref, uhi_ref, dinv_ref):
    deg = deg_ref[0, :] + deg_ref[1, :]
    dinv = lax.rsqrt(deg)
    u = dinv[:, None] * jnp.dot(x_ref[...], w_ref[...],
                                preferred_element_type=_f32)
    ulo_ref[...] = u[:, :DH]
    uhi_ref[...] = u[:, DH:]
    dinv_ref[...] = dinv


def _relu_mv_body(plo_ref, phi_ref, dinv_ref, b1_ref, w2_ref, u2_ref):
    ssum = jnp.concatenate(
        [plo_ref[0] + plo_ref[1], phi_ref[0] + phi_ref[1]], axis=1)
    dinv = dinv_ref[...]
    h = jnp.maximum(dinv[:, None] * ssum + b1_ref[...][None, :], 0.0)
    z = jnp.dot(h, w2_ref[...], preferred_element_type=_f32)
    u2_ref[...] = dinv * z[:, 0]


def _final_body(q_ref, dinv_ref, b2_ref, out_ref):
    v = dinv_ref[...] * (q_ref[0] + q_ref[1]) + b2_ref[0]
    out_ref[...] = jax.nn.sigmoid(v)


def kernel(x, edge_index, W1, b1, W2, b2):
    src = edge_index[0].astype(jnp.int32)
    dst = edge_index[1].astype(jnp.int32)

    npad = E_PAD - E
    pad_idx = (N + (jnp.arange(npad, dtype=jnp.int32) % (P - N)))
    src_p = jnp.concatenate([src, pad_idx])
    dst_p = jnp.concatenate([dst, pad_idx])
    idxp = jnp.stack(
        [src_p.reshape(NCHUNK, C), dst_p.reshape(NCHUNK, C)], axis=1)

    x_p = jnp.pad(x, ((0, P - N), (0, 0)))
    ones1 = jnp.ones((P,), _f32)
    zeros1 = jnp.zeros((P,), _f32)
    zeros2 = jnp.zeros((P, DH), _f32)

    deg_p = _deg_kernel(idxp, ones1, zeros1).reshape(2, P)

    u_lo, u_hi, dinv = pl.pallas_call(
        _mm_scale_body,
        out_shape=(jax.ShapeDtypeStruct((P, DH), _f32),
                   jax.ShapeDtypeStruct((P, DH), _f32),
                   jax.ShapeDtypeStruct((P,), _f32)),
    )(x_p, W1, deg_p)

    p_lo, p_hi = _agg1_kernel(idxp, u_lo, u_hi, zeros2)

    u2 = pl.pallas_call(
        _relu_mv_body,
        out_shape=jax.ShapeDtypeStruct((P,), _f32),
    )(p_lo.reshape(2, P, DH), p_hi.reshape(2, P, DH), dinv, b1, W2)

    part2 = _agg2_kernel(idxp, u2, zeros1).reshape(2, P)

    out_pad = pl.pallas_call(
        _final_body,
        out_shape=jax.ShapeDtypeStruct((P,), _f32),
    )(part2, dinv, b2)

    return out_pad[:N].reshape(N, 1)


# speedup vs baseline: 48.2565x; 1.0913x over previous
import functools

import jax
import jax.numpy as jnp
from jax import lax
from jax.experimental import pallas as pl
from jax.experimental.pallas import tpu as pltpu
from jax.experimental.pallas import tpu_sc as plsc

N = 10000
D = 128
DH = D // 2
P = 10112
E = 320000
C = 128
NC = 2
NS = 16
NW = NC * NS
CH = 80
E_PAD = NW * C * CH
NCHUNK = NW * CH
RPT = P // NS
NB = 5

_MESH = plsc.VectorSubcoreMesh(core_axis_name="c", subcore_axis_name="s")
_f32 = jnp.float32
_PIECES = ((0, C), (C, C), (2 * C, C), (3 * C, C), (4 * C, RPT - 4 * C))


def _fill_1d(ref, n, value):
    vec = jnp.full((16,), value, _f32)
    for j in range(n // 16):
        ref[pl.ds(j * 16, 16)] = vec
    if n % 16:
        ref[pl.ds(n - 16, 16)] = vec


@functools.partial(
    pl.kernel,
    out_type=jax.ShapeDtypeStruct((2 * P,), _f32),
    mesh=_MESH,
    scratch_types=[
        pltpu.VMEM((CH, 2, C), jnp.int32),
        pltpu.VMEM((C,), _f32),
        pltpu.VMEM((RPT,), _f32),
        pltpu.VMEM_SHARED((P,), _f32),
        pltpu.SemaphoreType.DMA,
    ],
)
def _deg_kernel(idxp_hbm, out_hbm, idx_v, ones_v, stage_v, deg_sh, sem):
    c = lax.axis_index("c")
    s = lax.axis_index("s")
    wid = s * NC + c
    lo = s * RPT
    cbase = wid * CH

    pltpu.sync_copy(idxp_hbm.at[pl.ds(cbase, CH)], idx_v)

    _fill_1d(ones_v, C, 1.0)
    _fill_1d(stage_v, RPT, 0.0)

    @pl.when(c == 0)
    def _():
        _fill_1d(stage_v, RPT, 1.0)

    pltpu.sync_copy(stage_v, deg_sh.at[pl.ds(lo, RPT)])
    plsc.subcore_barrier()

    def body(m, carry):
        pltpu.async_copy(ones_v, deg_sh.at[idx_v.at[m, 1]], sem, add=True)
        return carry

    lax.fori_loop(0, CH, body, 0)

    def drain(m, carry):
        pltpu.make_async_copy(ones_v, deg_sh.at[idx_v.at[0, 1]], sem).wait()
        return carry

    lax.fori_loop(0, CH, drain, 0)
    plsc.subcore_barrier()

    pltpu.sync_copy(deg_sh.at[pl.ds(lo, RPT)], stage_v)
    pltpu.sync_copy(stage_v, out_hbm.at[pl.ds(c * P + lo, RPT)])


@functools.partial(
    pl.kernel,
    out_type=(jax.ShapeDtypeStruct((2 * P, DH), _f32),
              jax.ShapeDtypeStruct((2 * P, DH), _f32)),
    mesh=_MESH,
    scratch_types=(
        [pltpu.VMEM((CH, 2, C), jnp.int32)]
        + [pltpu.VMEM((C, DH), _f32) for _ in range(NB)]
        + [pltpu.VMEM_SHARED((P, DH), _f32)]
        + [pltpu.SemaphoreType.DMA for _ in range(2 * NB)]
    ),
    compiler_params=pltpu.CompilerParams(use_tc_tiling_on_sc=False),
)
def _agg1_kernel(idxp_hbm, ulo_hbm, uhi_hbm, outlo_hbm, outhi_hbm,
                 idx_v, *bufs):
    rows = bufs[:NB]
    acc_sh = bufs[NB]
    gsem = bufs[NB + 1: 2 * NB + 1]
    ssem = bufs[2 * NB + 1:]

    c = lax.axis_index("c")
    s = lax.axis_index("s")
    wid = s * NC + c
    lo = s * RPT
    cbase = wid * CH

    pltpu.sync_copy(idxp_hbm.at[pl.ds(cbase, CH)], idx_v)

    for u_hbm, out_hbm in ((ulo_hbm, outlo_hbm), (uhi_hbm, outhi_hbm)):
        @pl.when(c == 0)
        def _():
            for off, ln in _PIECES:
                pltpu.sync_copy(u_hbm.at[pl.ds(lo + off, ln)],
                                rows[0].at[pl.ds(0, ln)])
                pltpu.sync_copy(rows[0].at[pl.ds(0, ln)],
                                acc_sh.at[pl.ds(lo + off, ln)])

        @pl.when(c == 1)
        def _():
            for j in range(C):
                for q in range(DH // 16):
                    rows[0][j, pl.ds(q * 16, 16)] = jnp.zeros((16,), _f32)
            for off, ln in _PIECES:
                pltpu.sync_copy(rows[0].at[pl.ds(0, ln)],
                                acc_sh.at[pl.ds(lo + off, ln)])

        plsc.subcore_barrier()

        pltpu.async_copy(u_hbm.at[idx_v.at[0, 0]], rows[0], gsem[0])
        pltpu.async_copy(u_hbm.at[idx_v.at[1, 0]], rows[1], gsem[1])
        pltpu.async_copy(u_hbm.at[idx_v.at[2, 0]], rows[2], gsem[2])

        def body(i, carry):
            k = i * NB
            for b in range(NB):
                m = k + b
                bn = (b + 3) % NB
                pltpu.make_async_copy(
                    u_hbm.at[pl.ds(0, C)], rows[b], gsem[b]).wait()
                pltpu.async_copy(
                    rows[b], acc_sh.at[idx_v.at[m, 1]], ssem[b], add=True)

                @pl.when(m + 3 < CH)
                def _():
                    @pl.when(m >= 2)
                    def _():
                        pltpu.make_async_copy(
                            rows[bn], acc_sh.at[idx_v.at[0, 1]],
                            ssem[bn]).wait()

                    pltpu.async_copy(
                        u_hbm.at[idx_v.at[m + 3, 0]], rows[bn], gsem[bn])
            return carry

        lax.fori_loop(0, CH // NB, body, 0)
        for b in range(NB):
            pltpu.make_async_copy(
                rows[b], acc_sh.at[idx_v.at[0, 1]], ssem[b]).wait()
        plsc.subcore_barrier()

        for off, ln in _PIECES:
            pltpu.sync_copy(acc_sh.at[pl.ds(lo + off, ln)],
                            rows[0].at[pl.ds(0, ln)])
            pltpu.sync_copy(rows[0].at[pl.ds(0, ln)],
                            out_hbm.at[pl.ds(c * P + lo + off, ln)])
        plsc.subcore_barrier()


@functools.partial(
    pl.kernel,
    out_type=jax.ShapeDtypeStruct((2 * P,), _f32),
    mesh=_MESH,
    scratch_types=[
        pltpu.VMEM((CH, 2, C), jnp.int32),
        pltpu.VMEM((C,), _f32),
        pltpu.VMEM((C,), _f32),
        pltpu.VMEM((P,), _f32),
        pltpu.VMEM((RPT,), _f32),
        pltpu.VMEM_SHARED((P,), _f32),
        pltpu.SemaphoreType.DMA,
        pltpu.SemaphoreType.DMA,
    ],
    compiler_params=pltpu.CompilerParams(needs_layout_passes=False),
)
def _agg2_kernel(idxp_hbm, u2_hbm, out_hbm,
                 idx_v, vals0, vals1, u2_v, stage_v, acc_sh, s0, s1):
    c = lax.axis_index("c")
    s = lax.axis_index("s")
    wid = s * NC + c
    lo = s * RPT
    cbase = wid * CH

    pltpu.sync_copy(u2_hbm, u2_v)
    pltpu.sync_copy(idxp_hbm.at[pl.ds(cbase, CH)], idx_v)

    @pl.when(c == 0)
    def _():
        pltpu.sync_copy(u2_v.at[pl.ds(lo, RPT)], acc_sh.at[pl.ds(lo, RPT)])

    @pl.when(c == 1)
    def _():
        _fill_1d(stage_v, RPT, 0.0)
        pltpu.sync_copy(stage_v, acc_sh.at[pl.ds(lo, RPT)])

    plsc.subcore_barrier()

    vals = (vals0, vals1)
    ssem = (s0, s1)

    def body(i, carry):
        k = i * 2
        for b in range(2):
            m = k + b
            for j in range(C // 16):
                sv = idx_v[m, 0, pl.ds(j * 16, 16)]
                vals[b][pl.ds(j * 16, 16)] = plsc.load_gather(u2_v, [sv])

            @pl.when(m >= 2)
            def _():
                pltpu.make_async_copy(
                    vals[b], acc_sh.at[idx_v.at[0, 1]], ssem[b]).wait()

            pltpu.async_copy(
                vals[b], acc_sh.at[idx_v.at[m, 1]], ssem[b], add=True)
        return carry

    lax.fori_loop(0, CH // 2, body, 0)
    for b in range(2):
        pltpu.make_async_copy(vals[b], acc_sh.at[idx_v.at[0, 1]], ssem[b]).wait()
    plsc.subcore_barrier()

    pltpu.sync_copy(acc_sh.at[pl.ds(lo, RPT)], stage_v)
    pltpu.sync_copy(stage_v, out_hbm.at[pl.ds(c * P + lo, RPT)])


def _mm_body(x_ref, w_ref, h_ref):
    h_ref[...] = jnp.dot(x_ref[...], w_ref[...], preferred_element_type=_f32)


def _scale_body(h_ref, deg_ref, ulo_ref, uhi_ref, dinv_ref):
    deg = deg_ref[0, :] + deg_ref[1, :]
    dinv = lax.rsqrt(deg)
    u = dinv[:, None] * h_ref[...]
    ulo_ref[...] = u[:, :DH]
    uhi_ref[...] = u[:, DH:]
    dinv_ref[...] = dinv


def _relu_mv_body(plo_ref, phi_ref, dinv_ref, b1_ref, w2_ref, u2_ref):
    ssum = jnp.concatenate(
        [plo_ref[0] + plo_ref[1], phi_ref[0] + phi_ref[1]], axis=1)
    dinv = dinv_ref[...]
    h = jnp.maximum(dinv[:, None] * ssum + b1_ref[...][None, :], 0.0)
    z = jnp.dot(h, w2_ref[...], preferred_element_type=_f32)
    u2_ref[...] = dinv * z[:, 0]


def _final_body(q_ref, dinv_ref, b2_ref, out_ref):
    v = dinv_ref[...] * (q_ref[0] + q_ref[1]) + b2_ref[0]
    out_ref[...] = jax.nn.sigmoid(v)


def kernel(x, edge_index, W1, b1, W2, b2):
    src = edge_index[0].astype(jnp.int32)
    dst = edge_index[1].astype(jnp.int32)

    npad = E_PAD - E
    pad_idx = (N + (jnp.arange(npad, dtype=jnp.int32) % (P - N)))
    src_p = jnp.concatenate([src, pad_idx])
    dst_p = jnp.concatenate([dst, pad_idx])
    idxp = jnp.stack(
        [src_p.reshape(NCHUNK, C), dst_p.reshape(NCHUNK, C)], axis=1)

    x_p = jnp.pad(x, ((0, P - N), (0, 0)))

    deg_p = _deg_kernel(idxp).reshape(2, P)
    h1 = pl.pallas_call(
        _mm_body,
        out_shape=jax.ShapeDtypeStruct((P, D), _f32),
    )(x_p, W1)

    u_lo, u_hi, dinv = pl.pallas_call(
        _scale_body,
        out_shape=(jax.ShapeDtypeStruct((P, DH), _f32),
                   jax.ShapeDtypeStruct((P, DH), _f32),
                   jax.ShapeDtypeStruct((P,), _f32)),
    )(h1, deg_p)

    p_lo, p_hi = _agg1_kernel(idxp, u_lo, u_hi)

    u2 = pl.pallas_call(
        _relu_mv_body,
        out_shape=jax.ShapeDtypeStruct((P,), _f32),
    )(p_lo.reshape(2, P, DH), p_hi.reshape(2, P, DH), dinv, b1, W2)

    part2 = _agg2_kernel(idxp, u2).reshape(2, P)

    out_pad = pl.pallas_call(
        _final_body,
        out_shape=jax.ShapeDtypeStruct((P,), _f32),
    )(part2, dinv, b2)

    return out_pad[:N].reshape(N, 1)


# speedup vs baseline: 52.8518x; 1.0952x over previous
import functools

import jax
import jax.numpy as jnp
from jax import lax
from jax.experimental import pallas as pl
from jax.experimental.pallas import tpu as pltpu
from jax.experimental.pallas import tpu_sc as plsc

N = 10000
D = 128
DH = D // 2
P = 10112
E = 320000
C = 128
NC = 2
NS = 16
NW = NC * NS
CH = 80
E_PAD = NW * C * CH
NCHUNK = NW * CH
RPT = P // NS
NB = 5

_MESH = plsc.VectorSubcoreMesh(core_axis_name="c", subcore_axis_name="s")
_f32 = jnp.float32
_PIECES = ((0, C), (C, C), (2 * C, C), (3 * C, C), (4 * C, RPT - 4 * C))


def _fill_1d(ref, n, value):
    vec = jnp.full((16,), value, _f32)
    for j in range(n // 16):
        ref[pl.ds(j * 16, 16)] = vec
    if n % 16:
        ref[pl.ds(n - 16, 16)] = vec


@functools.partial(
    pl.kernel,
    out_type=jax.ShapeDtypeStruct((2 * P,), _f32),
    mesh=_MESH,
    scratch_types=[
        pltpu.VMEM((CH, 2, C), jnp.int32),
        pltpu.VMEM((C,), _f32),
        pltpu.VMEM((RPT,), _f32),
        pltpu.VMEM_SHARED((P,), _f32),
        pltpu.SemaphoreType.DMA,
    ],
)
def _deg_kernel(idxp_hbm, out_hbm, idx_v, ones_v, stage_v, deg_sh, sem):
    c = lax.axis_index("c")
    s = lax.axis_index("s")
    wid = s * NC + c
    lo = s * RPT
    cbase = wid * CH

    pltpu.sync_copy(idxp_hbm.at[pl.ds(cbase, CH)], idx_v)

    _fill_1d(ones_v, C, 1.0)
    _fill_1d(stage_v, RPT, 0.0)

    @pl.when(c == 0)
    def _():
        _fill_1d(stage_v, RPT, 1.0)

    pltpu.sync_copy(stage_v, deg_sh.at[pl.ds(lo, RPT)])
    plsc.subcore_barrier()

    def body(m, carry):
        pltpu.async_copy(ones_v, deg_sh.at[idx_v.at[m, 1]], sem, add=True)
        return carry

    lax.fori_loop(0, CH, body, 0)

    def drain(m, carry):
        pltpu.make_async_copy(ones_v, deg_sh.at[idx_v.at[0, 1]], sem).wait()
        return carry

    lax.fori_loop(0, CH, drain, 0)
    plsc.subcore_barrier()

    pltpu.sync_copy(deg_sh.at[pl.ds(lo, RPT)], stage_v)
    pltpu.sync_copy(stage_v, out_hbm.at[pl.ds(c * P + lo, RPT)])


CHA = 2 * CH


@functools.partial(
    pl.kernel,
    out_type=jax.ShapeDtypeStruct((2 * P, DH), _f32),
    mesh=_MESH,
    scratch_types=(
        [pltpu.VMEM((CHA, 2, C), jnp.int32)]
        + [pltpu.VMEM((C, DH), _f32) for _ in range(NB)]
        + [pltpu.VMEM_SHARED((P, DH), _f32)]
        + [pltpu.SemaphoreType.DMA for _ in range(2 * NB)]
    ),
    compiler_params=pltpu.CompilerParams(use_tc_tiling_on_sc=False),
)
def _agg1_kernel(idxp_hbm, ub_hbm, out_hbm, idx_v, *bufs):
    rows = bufs[:NB]
    acc_sh = bufs[NB]
    gsem = bufs[NB + 1: 2 * NB + 1]
    ssem = bufs[2 * NB + 1:]

    c = lax.axis_index("c")
    s = lax.axis_index("s")
    lo = s * RPT
    cbase = s * CHA
    u_hbm = ub_hbm.at[c]

    pltpu.sync_copy(idxp_hbm.at[pl.ds(cbase, CHA)], idx_v)

    for off, ln in _PIECES:
        pltpu.sync_copy(u_hbm.at[pl.ds(lo + off, ln)],
                        rows[0].at[pl.ds(0, ln)])
        pltpu.sync_copy(rows[0].at[pl.ds(0, ln)],
                        acc_sh.at[pl.ds(lo + off, ln)])

    plsc.subcore_barrier()

    pltpu.async_copy(u_hbm.at[idx_v.at[0, 0]], rows[0], gsem[0])
    pltpu.async_copy(u_hbm.at[idx_v.at[1, 0]], rows[1], gsem[1])
    pltpu.async_copy(u_hbm.at[idx_v.at[2, 0]], rows[2], gsem[2])

    def body(i, carry):
        k = i * NB
        for b in range(NB):
            m = k + b
            bn = (b + 3) % NB
            pltpu.make_async_copy(
                u_hbm.at[pl.ds(0, C)], rows[b], gsem[b]).wait()
            pltpu.async_copy(
                rows[b], acc_sh.at[idx_v.at[m, 1]], ssem[b], add=True)

            @pl.when(m + 3 < CHA)
            def _():
                @pl.when(m >= 2)
                def _():
                    pltpu.make_async_copy(
                        rows[bn], acc_sh.at[idx_v.at[0, 1]],
                        ssem[bn]).wait()

                pltpu.async_copy(
                    u_hbm.at[idx_v.at[m + 3, 0]], rows[bn], gsem[bn])
        return carry

    lax.fori_loop(0, CHA // NB, body, 0)
    for b in range(NB):
        pltpu.make_async_copy(
            rows[b], acc_sh.at[idx_v.at[0, 1]], ssem[b]).wait()
    plsc.subcore_barrier()

    for off, ln in _PIECES:
        pltpu.sync_copy(acc_sh.at[pl.ds(lo + off, ln)],
                        rows[0].at[pl.ds(0, ln)])
        pltpu.sync_copy(rows[0].at[pl.ds(0, ln)],
                        out_hbm.at[pl.ds(c * P + lo + off, ln)])


@functools.partial(
    pl.kernel,
    out_type=jax.ShapeDtypeStruct((2 * P,), _f32),
    mesh=_MESH,
    scratch_types=[
        pltpu.VMEM((CH, 2, C), jnp.int32),
        pltpu.VMEM((C,), _f32),
        pltpu.VMEM((C,), _f32),
        pltpu.VMEM((P,), _f32),
        pltpu.VMEM((RPT,), _f32),
        pltpu.VMEM_SHARED((P,), _f32),
        pltpu.SemaphoreType.DMA,
        pltpu.SemaphoreType.DMA,
    ],
    compiler_params=pltpu.CompilerParams(needs_layout_passes=False),
)
def _agg2_kernel(idxp_hbm, u2_hbm, out_hbm,
                 idx_v, vals0, vals1, u2_v, stage_v, acc_sh, s0, s1):
    c = lax.axis_index("c")
    s = lax.axis_index("s")
    wid = s * NC + c
    lo = s * RPT
    cbase = wid * CH

    pltpu.sync_copy(u2_hbm, u2_v)
    pltpu.sync_copy(idxp_hbm.at[pl.ds(cbase, CH)], idx_v)

    @pl.when(c == 0)
    def _():
        pltpu.sync_copy(u2_v.at[pl.ds(lo, RPT)], acc_sh.at[pl.ds(lo, RPT)])

    @pl.when(c == 1)
    def _():
        _fill_1d(stage_v, RPT, 0.0)
        pltpu.sync_copy(stage_v, acc_sh.at[pl.ds(lo, RPT)])

    plsc.subcore_barrier()

    vals = (vals0, vals1)
    ssem = (s0, s1)

    def body(i, carry):
        k = i * 2
        for b in range(2):
            m = k + b
            for j in range(C // 16):
                sv = idx_v[m, 0, pl.ds(j * 16, 16)]
                vals[b][pl.ds(j * 16, 16)] = plsc.load_gather(u2_v, [sv])

            @pl.when(m >= 2)
            def _():
                pltpu.make_async_copy(
                    vals[b], acc_sh.at[idx_v.at[0, 1]], ssem[b]).wait()

            pltpu.async_copy(
                vals[b], acc_sh.at[idx_v.at[m, 1]], ssem[b], add=True)
        return carry

    lax.fori_loop(0, CH // 2, body, 0)
    for b in range(2):
        pltpu.make_async_copy(vals[b], acc_sh.at[idx_v.at[0, 1]], ssem[b]).wait()
    plsc.subcore_barrier()

    pltpu.sync_copy(acc_sh.at[pl.ds(lo, RPT)], stage_v)
    pltpu.sync_copy(stage_v, out_hbm.at[pl.ds(c * P + lo, RPT)])


def _mm_body(x_ref, w_ref, h_ref):
    h_ref[...] = jnp.dot(x_ref[...], w_ref[...], preferred_element_type=_f32)


def _scale_body(h_ref, deg_ref, ub_ref, dinv_ref):
    deg = deg_ref[0, :] + deg_ref[1, :]
    dinv = lax.rsqrt(deg)
    u = dinv[:, None] * h_ref[...]
    ub_ref[0] = u[:, :DH]
    ub_ref[1] = u[:, DH:]
    dinv_ref[...] = dinv


def _relu_mv_body(p_ref, dinv_ref, b1_ref, w2_ref, u2_ref):
    ssum = jnp.concatenate([p_ref[0], p_ref[1]], axis=1)
    dinv = dinv_ref[...]
    h = jnp.maximum(dinv[:, None] * ssum + b1_ref[...][None, :], 0.0)
    z = jnp.dot(h, w2_ref[...], preferred_element_type=_f32)
    u2_ref[...] = dinv * z[:, 0]


def _final_body(q_ref, dinv_ref, b2_ref, out_ref):
    v = dinv_ref[...] * (q_ref[0] + q_ref[1]) + b2_ref[0]
    out_ref[...] = jax.nn.sigmoid(v)


def kernel(x, edge_index, W1, b1, W2, b2):
    src = edge_index[0].astype(jnp.int32)
    dst = edge_index[1].astype(jnp.int32)

    npad = E_PAD - E
    pad_idx = (N + (jnp.arange(npad, dtype=jnp.int32) % (P - N)))
    src_p = jnp.concatenate([src, pad_idx])
    dst_p = jnp.concatenate([dst, pad_idx])
    idxp = jnp.stack(
        [src_p.reshape(NCHUNK, C), dst_p.reshape(NCHUNK, C)], axis=1)

    x_p = jnp.pad(x, ((0, P - N), (0, 0)))

    deg_p = _deg_kernel(idxp).reshape(2, P)
    h1 = pl.pallas_call(
        _mm_body,
        out_shape=jax.ShapeDtypeStruct((P, D), _f32),
    )(x_p, W1)

    u_both, dinv = pl.pallas_call(
        _scale_body,
        out_shape=(jax.ShapeDtypeStruct((2, P, DH), _f32),
                   jax.ShapeDtypeStruct((P,), _f32)),
    )(h1, deg_p)

    p_both = _agg1_kernel(idxp, u_both)

    u2 = pl.pallas_call(
        _relu_mv_body,
        out_shape=jax.ShapeDtypeStruct((P,), _f32),
    )(p_both.reshape(2, P, DH), dinv, b1, W2)

    part2 = _agg2_kernel(idxp, u2).reshape(2, P)

    out_pad = pl.pallas_call(
        _final_body,
        out_shape=jax.ShapeDtypeStruct((P,), _f32),
    )(part2, dinv, b2)

    return out_pad[:N].reshape(N, 1)
